# baseline (speedup 1.0000x reference)
import functools

import jax
import jax.numpy as jnp
from jax import lax
from jax.experimental import pallas as pl
from jax.experimental.pallas import tpu as pltpu
from jax.experimental.pallas import tpu_sc as plsc

NROWS = 1000000
HIDDEN = 32
BATCH = 4096

_INFO = plsc.get_sparse_core_info()
_NC = _INFO.num_cores
_NS = _INFO.num_subcores
_L = _INFO.num_lanes
_NW = _NC * _NS
_BPW = BATCH // _NW
_NCH = _BPW // _L


def _score_kernel(hidx_hbm, ridx_hbm, tidx_hbm, ent_hbm, rel_hbm, out_hbm,
                  idx_h, idx_r, idx_t,
                  exp_h, exp_r, exp_t,
                  dst_h, dst_r, dst_t, out_v,
                  sem_h, sem_r, sem_t):
    wid = lax.axis_index("s") * _NC + lax.axis_index("c")
    base = wid * _BPW

    pltpu.sync_copy(hidx_hbm.at[pl.ds(base, _BPW)], idx_h)
    pltpu.sync_copy(ridx_hbm.at[pl.ds(base, _BPW)], idx_r)
    pltpu.sync_copy(tidx_hbm.at[pl.ds(base, _BPW)], idx_t)

    def expand(k, carry):
        eh = idx_h[pl.ds(k * _L, _L)]
        er = idx_r[pl.ds(k * _L, _L)]
        et = idx_t[pl.ds(k * _L, _L)]
        for c in range(HIDDEN):
            off = c * NROWS
            exp_h[c, pl.ds(k * _L, _L)] = eh + off
            exp_r[c, pl.ds(k * _L, _L)] = er + off
            exp_t[c, pl.ds(k * _L, _L)] = et + off
        return carry

    lax.fori_loop(0, _NCH, expand, 0)

    copies = []
    for c in range(HIDDEN):
        copies.append(pltpu.async_copy(ent_hbm.at[exp_h.at[c]], dst_h.at[c], sem_h))
        copies.append(pltpu.async_copy(rel_hbm.at[exp_r.at[c]], dst_r.at[c], sem_r))
        copies.append(pltpu.async_copy(ent_hbm.at[exp_t.at[c]], dst_t.at[c], sem_t))
    for cp in copies:
        cp.wait()

    def score(k, carry):
        acc = jnp.zeros((_L,), jnp.float32)
        for c in range(HIDDEN):
            d = (dst_h[c, pl.ds(k * _L, _L)]
                 + dst_r[c, pl.ds(k * _L, _L)]
                 - dst_t[c, pl.ds(k * _L, _L)])
            acc = acc + jnp.abs(d)
        out_v[pl.ds(k * _L, _L)] = acc
        return carry

    lax.fori_loop(0, _NCH, score, 0)

    pltpu.sync_copy(out_v, out_hbm.at[pl.ds(base, _BPW)])


@jax.jit
def _scores(hidx, ridx, tidx, ent_flat, rel_flat):
    mesh = plsc.VectorSubcoreMesh(core_axis_name="c", subcore_axis_name="s")
    kern = functools.partial(
        pl.kernel,
        mesh=mesh,
        out_type=jax.ShapeDtypeStruct((BATCH,), jnp.float32),
        scratch_types=[
            pltpu.VMEM((_BPW,), jnp.int32),
            pltpu.VMEM((_BPW,), jnp.int32),
            pltpu.VMEM((_BPW,), jnp.int32),
            pltpu.VMEM((HIDDEN, _BPW), jnp.int32),
            pltpu.VMEM((HIDDEN, _BPW), jnp.int32),
            pltpu.VMEM((HIDDEN, _BPW), jnp.int32),
            pltpu.VMEM((HIDDEN, _BPW), jnp.float32),
            pltpu.VMEM((HIDDEN, _BPW), jnp.float32),
            pltpu.VMEM((HIDDEN, _BPW), jnp.float32),
            pltpu.VMEM((_BPW,), jnp.float32),
            pltpu.SemaphoreType.DMA,
            pltpu.SemaphoreType.DMA,
            pltpu.SemaphoreType.DMA,
        ],
    )(_score_kernel)
    return kern(hidx, ridx, tidx, ent_flat, rel_flat)


def kernel(sample, entity_embedding, relation_embedding):
    ent_flat = entity_embedding.T.reshape(-1)
    rel_flat = relation_embedding.T.reshape(-1)
    out = _scores(sample[:, 0], sample[:, 1], sample[:, 2],
                  ent_flat, rel_flat)
    return out.reshape(BATCH, 1)


# speedup vs baseline: 50.3801x; 50.3801x over previous
import functools

import jax
import jax.numpy as jnp
from jax import lax
from jax.experimental import pallas as pl
from jax.experimental.pallas import tpu as pltpu
from jax.experimental.pallas import tpu_sc as plsc

NROWS = 1000000
HIDDEN = 32
BATCH = 4096

_INFO = plsc.get_sparse_core_info()
_NC = _INFO.num_cores
_NS = _INFO.num_subcores
_L = _INFO.num_lanes
_NW = _NC * _NS
_BPW = BATCH // _NW
_NB = 4


def _fetch(tab, e, blk, b, sem):
    e128 = pl.multiple_of((e >> 7) << 7, 128)
    return pltpu.async_copy(tab.at[:, pl.ds(e128, 128)], blk.at[b], sem[b])


def _drain(tab, blk, b, sem):
    pltpu.make_async_copy(tab.at[:, pl.ds(0, 128)], blk.at[b], sem[b]).wait()


def _score_kernel(hidx_hbm, ridx_hbm, tidx_hbm, ent_hbm, rel_hbm, out_hbm,
                  idx_h, idx_r, idx_t,
                  blk_h, blk_r, blk_t, out_v,
                  sem_h, sem_r, sem_t):
    wid = lax.axis_index("s") * _NC + lax.axis_index("c")
    base = wid * _BPW

    pltpu.sync_copy(hidx_hbm.at[pl.ds(base, _BPW)], idx_h.at[pl.ds(0, _BPW)])
    pltpu.sync_copy(ridx_hbm.at[pl.ds(base, _BPW)], idx_r.at[pl.ds(0, _BPW)])
    pltpu.sync_copy(tidx_hbm.at[pl.ds(base, _BPW)], idx_t.at[pl.ds(0, _BPW)])

    iota = lax.iota(jnp.int32, _L)

    def issue(eh, er, et, b):
        _fetch(ent_hbm, eh, blk_h, b, sem_h)
        _fetch(rel_hbm, er, blk_r, b, sem_r)
        _fetch(ent_hbm, et, blk_t, b, sem_t)

    ch0_h = idx_h[pl.ds(0, _L)]
    ch0_r = idx_r[pl.ds(0, _L)]
    ch0_t = idx_t[pl.ds(0, _L)]
    for b in range(_NB):
        issue(ch0_h[b], ch0_r[b], ch0_t[b], b)

    def sample_score(eh, er, et, b):
        gh = ((eh & 127) >> 4) << 4
        gr = ((er & 127) >> 4) << 4
        gt = ((et & 127) >> 4) << 4
        lh = jnp.full((_L,), eh & 15, jnp.int32)
        lr = jnp.full((_L,), er & 15, jnp.int32)
        lt = jnp.full((_L,), et & 15, jnp.int32)

        def chan(c, acc):
            h = jnp.take(blk_h[b, c, pl.ds(gh, _L)], lh)
            r = jnp.take(blk_r[b, c, pl.ds(gr, _L)], lr)
            t = jnp.take(blk_t[b, c, pl.ds(gt, _L)], lt)
            return acc + jnp.abs(h + r - t)

        return lax.fori_loop(0, HIDDEN, chan, jnp.zeros((_L,), jnp.float32))

    def chunk(k, carry):
        cur_h = idx_h[pl.ds(k * _L, _L)]
        cur_r = idx_r[pl.ds(k * _L, _L)]
        cur_t = idx_t[pl.ds(k * _L, _L)]
        nxt_h = idx_h[pl.ds(k * _L + _L, _L)]
        nxt_r = idx_r[pl.ds(k * _L + _L, _L)]
        nxt_t = idx_t[pl.ds(k * _L + _L, _L)]
        outacc = jnp.zeros((_L,), jnp.float32)
        for j in range(_L):
            b = j % _NB
            i = k * _L + j
            _drain(ent_hbm, blk_h, b, sem_h)
            _drain(rel_hbm, blk_r, b, sem_r)
            _drain(ent_hbm, blk_t, b, sem_t)
            s = sample_score(cur_h[j], cur_r[j], cur_t[j], b)
            outacc = jnp.where(iota == j, s, outacc)

            @pl.when(i + _NB < _BPW)
            def _():
                if j + _NB < _L:
                    issue(cur_h[j + _NB], cur_r[j + _NB], cur_t[j + _NB], b)
                else:
                    issue(nxt_h[j + _NB - _L], nxt_r[j + _NB - _L],
                          nxt_t[j + _NB - _L], b)

        out_v[pl.ds(k * _L, _L)] = outacc
        return carry

    lax.fori_loop(0, _BPW // _L, chunk, 0)

    pltpu.sync_copy(out_v, out_hbm.at[pl.ds(base, _BPW)])


@jax.jit
def _scores(hidx, ridx, tidx, ent_t, rel_t):
    mesh = plsc.VectorSubcoreMesh(core_axis_name="c", subcore_axis_name="s")
    kern = functools.partial(
        pl.kernel,
        mesh=mesh,
        compiler_params=pltpu.CompilerParams(use_tc_tiling_on_sc=True),
        out_type=jax.ShapeDtypeStruct((BATCH,), jnp.float32),
        scratch_types=[
            pltpu.VMEM((_BPW + _L,), jnp.int32),
            pltpu.VMEM((_BPW + _L,), jnp.int32),
            pltpu.VMEM((_BPW + _L,), jnp.int32),
            pltpu.VMEM((_NB, HIDDEN, 128), jnp.float32),
            pltpu.VMEM((_NB, HIDDEN, 128), jnp.float32),
            pltpu.VMEM((_NB, HIDDEN, 128), jnp.float32),
            pltpu.VMEM((_BPW,), jnp.float32),
            [pltpu.SemaphoreType.DMA] * _NB,
            [pltpu.SemaphoreType.DMA] * _NB,
            [pltpu.SemaphoreType.DMA] * _NB,
        ],
    )(_score_kernel)
    return kern(hidx, ridx, tidx, ent_t, rel_t)


def kernel(sample, entity_embedding, relation_embedding):
    out = _scores(sample[:, 0], sample[:, 1], sample[:, 2],
                  entity_embedding.T, relation_embedding.T)
    return out.reshape(BATCH, 1)
